# baseline (speedup 1.0000x reference)
import jax
import jax.numpy as jnp
from jax.experimental import pallas as pl
from jax.experimental.pallas import tpu as pltpu


def _pick_block(n, target=200):
    for b in range(min(n, target), 7, -1):
        if n % b == 0 and b % 8 == 0:
            return b
    return n


def _dot(a, b):
    return jax.lax.dot_general(
        a, b, (((1,), (0,)), ((), ())), preferred_element_type=jnp.float32
    )


def _xw_kernel(x_ref, w_ref, b_ref, o_ref):
    o_ref[...] = (_dot(x_ref[...], w_ref[...]) + b_ref[...]).astype(jnp.bfloat16)


def _make_fused(nb, nr, stride, br):
    def _res_slot(j):
        return j // stride

    def _is_res(j):
        return jnp.logical_and(j % stride == 0, _res_slot(j) < nr)

    def _fused(adj_ref, xw_ref, w2_ref, b2_ref, out_ref, hw_ref, res_ref):
        g = pl.program_id(0)

        @pl.when(g < nb)
        def _phase1():
            a16 = adj_ref[...].astype(jnp.bfloat16)
            h = jnp.maximum(_dot(a16, xw_ref[...]), 0.0).astype(jnp.bfloat16)
            hwb = _dot(h, w2_ref[...]) + b2_ref[...]
            hw_ref[pl.ds(pl.multiple_of(g * br, br), br), :] = hwb.astype(
                jnp.bfloat16
            )

            @pl.when(_is_res(g))
            def _save():
                res_ref[_res_slot(g)] = a16

        @pl.when(g >= nb)
        def _phase2():
            j = g - nb

            @pl.when(_is_res(j))
            def _resident():
                out_ref[...] = _dot(res_ref[_res_slot(j)], hw_ref[...])

            @pl.when(jnp.logical_not(_is_res(j)))
            def _streamed():
                a16 = adj_ref[...].astype(jnp.bfloat16)
                out_ref[...] = _dot(a16, hw_ref[...])

    return _fused


def kernel(x, Adj, W1, b1, W2, b2):
    n, _ = x.shape
    d_hid = W1.shape[1]
    d_out = W2.shape[1]
    br = _pick_block(n)
    nb = n // br
    slab_bytes = ((br + 15) // 16) * 16 * n * 2
    nr = min(nb, (37 * 1024 * 1024) // slab_bytes)
    stride = max(2, nb // max(nr, 1))
    nr = min(nr, (nb + stride - 1) // stride)
    b1r = b1.reshape(1, d_hid)
    b2r = b2.reshape(1, d_out)

    xw = pl.pallas_call(
        _xw_kernel,
        out_shape=jax.ShapeDtypeStruct((n, d_hid), jnp.bfloat16),
    )(x, W1, b1r)

    def _is_res(j):
        return jnp.logical_and(j % stride == 0, j // stride < nr)

    def adj_idx(g):
        j = g - nb
        p2 = jnp.where(_is_res(j), jnp.minimum(j + 1, nb - 1), j)
        return (jnp.where(g < nb, g, p2), 0)

    def out_idx(g):
        return (jnp.where(g < nb, 0, g - nb), 0)

    out = pl.pallas_call(
        _make_fused(nb, nr, stride, br),
        grid=(2 * nb,),
        in_specs=[
            pl.BlockSpec((br, n), adj_idx),
            pl.BlockSpec((n, d_hid), lambda g: (0, 0)),
            pl.BlockSpec((d_hid, d_out), lambda g: (0, 0)),
            pl.BlockSpec((1, d_out), lambda g: (0, 0)),
        ],
        out_specs=pl.BlockSpec((br, d_out), out_idx),
        out_shape=jax.ShapeDtypeStruct((n, d_out), jnp.float32),
        scratch_shapes=[
            pltpu.VMEM((n, d_hid), jnp.bfloat16),
            pltpu.VMEM((max(nr, 1), br, n), jnp.bfloat16),
        ],
        compiler_params=pltpu.CompilerParams(
            dimension_semantics=("arbitrary",),
            vmem_limit_bytes=64 * 1024 * 1024,
        ),
    )(Adj, xw, W2.astype(jnp.bfloat16), b2r)
    return out


# speedup vs baseline: 1.0475x; 1.0475x over previous
import jax
import jax.numpy as jnp
from jax.experimental import pallas as pl
from jax.experimental.pallas import tpu as pltpu


def _pick_block(n, target=200):
    for b in range(min(n, target), 7, -1):
        if n % b == 0 and b % 8 == 0:
            return b
    return n


def _dot(a, b):
    return jax.lax.dot_general(
        a, b, (((1,), (0,)), ((), ())), preferred_element_type=jnp.float32
    )


def _xw_kernel(x_ref, w_ref, b_ref, o_ref):
    o_ref[...] = (_dot(x_ref[...], w_ref[...]) + b_ref[...]).astype(jnp.bfloat16)


def _make_fused(nb, nr, br):
    def _fused(adj_ref, xw_ref, w2_ref, b2_ref, out_ref, hw_ref, res_ref):
        g = pl.program_id(0)

        @pl.when(g < nb)
        def _phase1():
            a = adj_ref[...].astype(jnp.bfloat16)
            h = jnp.maximum(_dot(a, xw_ref[...]), 0.0).astype(jnp.bfloat16)
            hwb = (_dot(h, w2_ref[...]) + b2_ref[...]).astype(jnp.bfloat16)
            hw_ref[pl.ds(pl.multiple_of(g * br, br), br), :] = hwb

            @pl.when(g < nr)
            def _save():
                res_ref[pl.ds(pl.multiple_of(g * br, br), br), :] = a

        @pl.when(g >= nb)
        def _phase2():
            j = g - nb

            @pl.when(j < nr)
            def _resident():
                a = res_ref[pl.ds(pl.multiple_of(j * br, br), br), :]
                out_ref[...] = _dot(a, hw_ref[...])

            @pl.when(j >= nr)
            def _streamed():
                a = adj_ref[...].astype(jnp.bfloat16)
                out_ref[...] = _dot(a, hw_ref[...])

    return _fused


def kernel(x, Adj, W1, b1, W2, b2):
    n, _ = x.shape
    d_hid = W1.shape[1]
    d_out = W2.shape[1]
    br = _pick_block(n)
    nb = n // br
    nr = min(nb, (37 * 1024 * 1024) // (br * n * 2))
    b1r = b1.reshape(1, d_hid)
    b2r = b2.reshape(1, d_out)

    xw = pl.pallas_call(
        _xw_kernel,
        out_shape=jax.ShapeDtypeStruct((n, d_hid), jnp.bfloat16),
    )(x, W1, b1r)

    def adj_idx(g):
        return (jnp.where(g < nb, g, jnp.where(g < nb + nr, nb - 1, g - nb)), 0)

    def out_idx(g):
        return (jnp.where(g < nb, 0, g - nb), 0)

    out = pl.pallas_call(
        _make_fused(nb, nr, br),
        grid=(2 * nb,),
        in_specs=[
            pl.BlockSpec((br, n), adj_idx),
            pl.BlockSpec((n, d_hid), lambda g: (0, 0)),
            pl.BlockSpec((d_hid, d_out), lambda g: (0, 0)),
            pl.BlockSpec((1, d_out), lambda g: (0, 0)),
        ],
        out_specs=pl.BlockSpec((br, d_out), out_idx),
        out_shape=jax.ShapeDtypeStruct((n, d_out), jnp.float32),
        scratch_shapes=[
            pltpu.VMEM((n, d_hid), jnp.bfloat16),
            pltpu.VMEM((max(nr, 1) * br, n), jnp.bfloat16),
        ],
        compiler_params=pltpu.CompilerParams(
            dimension_semantics=("arbitrary",),
            vmem_limit_bytes=64 * 1024 * 1024,
        ),
    )(Adj, xw, W2.astype(jnp.bfloat16), b2r)
    return out
